# baseline (speedup 1.0000x reference)
import functools
import math

import jax
import jax.numpy as jnp
import numpy as np
from jax import lax
from jax.experimental import pallas as pl
from jax.experimental.pallas import tpu as pltpu
from jax.experimental.pallas import tpu_sc as plsc


_D_MODEL = 1024
_SEQ_LEN = 4096
_BATCH = 4
_ROWS = _BATCH * _SEQ_LEN
_NW = 32
_ROWS_PER_W = _ROWS // _NW
_CH = 16
_CHUNK = _CH * _D_MODEL
_N_CHUNKS = _ROWS_PER_W // _CH
_UNROLL = 8
_LANES = 16


def _pe_table():
    pe = np.zeros((_SEQ_LEN, _D_MODEL), dtype=np.float32)
    position = np.arange(0, _SEQ_LEN)[:, np.newaxis]
    half = _D_MODEL // 2
    scale = -np.log(10000.0) / (half - 1)
    div = np.exp(np.arange(0, half) * scale)
    pe[:, :half] = np.sin(position * div)
    pe[:, half:] = np.cos(position * div)
    return jnp.asarray(pe.reshape(-1))


def _sc_body(x_hbm, pe_hbm, out_hbm, xbuf, pebuf):
    wid = lax.axis_index("s") * 2 + lax.axis_index("c")
    row0 = wid * _ROWS_PER_W
    x_base = row0 * _D_MODEL
    pe_base = (row0 % _SEQ_LEN) * _D_MODEL

    def chunk(c, carry):
        off = c * _CHUNK
        pltpu.sync_copy(x_hbm.at[pl.ds(x_base + off, _CHUNK)], xbuf)
        pltpu.sync_copy(pe_hbm.at[pl.ds(pe_base + off, _CHUNK)], pebuf)

        def add(k, carry2):
            base = k * (_LANES * _UNROLL)
            for u in range(_UNROLL):
                o = base + u * _LANES
                xbuf[pl.ds(o, _LANES)] = (
                    xbuf[pl.ds(o, _LANES)] + pebuf[pl.ds(o, _LANES)]
                )
            return carry2

        lax.fori_loop(0, _CHUNK // (_LANES * _UNROLL), add, 0)
        pltpu.sync_copy(xbuf, out_hbm.at[pl.ds(x_base + off, _CHUNK)])
        return carry

    lax.fori_loop(0, _N_CHUNKS, chunk, 0)


@functools.partial(
    pl.kernel,
    out_type=jax.ShapeDtypeStruct((_ROWS * _D_MODEL,), jnp.float32),
    mesh=plsc.VectorSubcoreMesh(core_axis_name="c", subcore_axis_name="s"),
    scratch_types=[
        pltpu.VMEM((_CHUNK,), jnp.float32),
        pltpu.VMEM((_CHUNK,), jnp.float32),
    ],
)
def _sc_add(x_hbm, pe_hbm, out_hbm, xbuf, pebuf):
    _sc_body(x_hbm, pe_hbm, out_hbm, xbuf, pebuf)


def kernel(inputs):
    batch, seq_len, d_model = inputs.shape
    assert (batch, seq_len, d_model) == (_BATCH, _SEQ_LEN, _D_MODEL)
    out = _sc_add(inputs.reshape(-1), _pe_table())
    return out.reshape(batch, seq_len, d_model)


# speedup vs baseline: 6.7608x; 6.7608x over previous
import math

import jax
import jax.numpy as jnp
import numpy as np
from jax.experimental import pallas as pl


_D_MODEL = 1024
_HALF = _D_MODEL // 2
_T = 32
_BLK = 2048
_SEQ = 4096


def _factor_tables():
    scale = -np.log(10000.0) / (_HALF - 1)
    div = np.exp(np.arange(_HALF) * scale)
    alpha = (np.arange(_SEQ // _T) * _T)[:, None] * div
    beta = np.arange(_T)[:, None] * div
    return (
        jnp.asarray(np.sin(alpha), dtype=jnp.float32),
        jnp.asarray(np.cos(alpha), dtype=jnp.float32),
        jnp.asarray(np.sin(beta), dtype=jnp.float32),
        jnp.asarray(np.cos(beta), dtype=jnp.float32),
    )


def _pe_add_body(x_ref, sa_ref, ca_ref, sb_ref, cb_ref, o_ref):
    q_grp = _BLK // _T
    sa = sa_ref[...].reshape(q_grp, 1, _HALF)
    ca = ca_ref[...].reshape(q_grp, 1, _HALF)
    sb = sb_ref[...].reshape(1, _T, _HALF)
    cb = cb_ref[...].reshape(1, _T, _HALF)
    pe_sin = (sa * cb + ca * sb).reshape(_BLK, _HALF)
    pe_cos = (ca * cb - sa * sb).reshape(_BLK, _HALF)
    o_ref[:, :_HALF] = x_ref[:, :_HALF] + pe_sin
    o_ref[:, _HALF:] = x_ref[:, _HALF:] + pe_cos


def kernel(inputs):
    batch, seq_len, d_model = inputs.shape
    assert d_model == _D_MODEL and seq_len == _SEQ
    rows = batch * seq_len
    x = inputs.reshape(rows, d_model)
    sa, ca, sb, cb = _factor_tables()
    q_grp = _BLK // _T
    blocks_per_batch = seq_len // _BLK
    out = pl.pallas_call(
        _pe_add_body,
        grid=(rows // _BLK,),
        in_specs=[
            pl.BlockSpec((_BLK, d_model), lambda i: (i, 0)),
            pl.BlockSpec((q_grp, _HALF), lambda i: (i % blocks_per_batch, 0)),
            pl.BlockSpec((q_grp, _HALF), lambda i: (i % blocks_per_batch, 0)),
            pl.BlockSpec((_T, _HALF), lambda i: (0, 0)),
            pl.BlockSpec((_T, _HALF), lambda i: (0, 0)),
        ],
        out_specs=pl.BlockSpec((_BLK, d_model), lambda i: (i, 0)),
        out_shape=jax.ShapeDtypeStruct((rows, d_model), inputs.dtype),
    )(x, sa, ca, sb, cb)
    return out.reshape(batch, seq_len, d_model)
